# baseline (speedup 1.0000x reference)
import functools

import jax
import jax.numpy as jnp
from jax import lax
from jax.experimental import pallas as pl
from jax.experimental.pallas import tpu as pltpu
from jax.experimental.pallas import tpu_sc as plsc

B, R, KT, C, H = 4096, 8, 2, 8, 128
K = KT + 1
BB = 256
NCH = 2
BC = B // NCH
NC, NS, L = 2, 16, 16
NW = NC * NS
BPW = BC // NW


def _sc_gather_body(off, tgt_hbm, c0, c1, c2, c3, o0, o1, o2, o3,
                    tgt_v, idx_v, r0, r1, r2, r3, sem):
    wid = lax.axis_index("s") * NC + lax.axis_index("c")
    base = wid * BPW
    pltpu.sync_copy(tgt_hbm.at[pl.ds(off + base, BPW)], tgt_v)
    for i in range(BPW // L):
        tv = tgt_v[pl.ds(L * i, L)]
        bb = lax.iota(jnp.int32, L) + (off + base + L * i)
        idx_v[pl.ds(L * i, L)] = bb * C + tv
    copies = [pltpu.async_copy(c.at[idx_v], r, sem)
              for c, r in ((c0, r0), (c1, r1), (c2, r2), (c3, r3))]
    for cp in copies:
        cp.wait()
    for r, o in ((r0, o0), (r1, o1), (r2, o2), (r3, o3)):
        pltpu.sync_copy(r, o.at[pl.ds(base, BPW)])


def _sc_gather(off, targets, cands):
    row = jax.ShapeDtypeStruct((BC, H), jnp.float32)
    fn = pl.kernel(
        functools.partial(_sc_gather_body, off),
        mesh=plsc.VectorSubcoreMesh(core_axis_name="c", subcore_axis_name="s"),
        out_type=[row] * 4,
        scratch_types=[
            pltpu.VMEM((BPW,), jnp.int32),
            pltpu.VMEM((BPW,), jnp.int32),
            pltpu.VMEM((BPW, H), jnp.float32),
            pltpu.VMEM((BPW, H), jnp.float32),
            pltpu.VMEM((BPW, H), jnp.float32),
            pltpu.VMEM((BPW, H), jnp.float32),
            pltpu.SemaphoreType.DMA,
        ],
    )
    return fn(targets, *cands)


def _fused_body(*refs):
    outs = refs[-4:]
    ones_h = jnp.full((H, H), -1.0 / (K * H), dtype=jnp.float32)

    for a in range(4):
        o_ref, te_ref, ch_ref = refs[3 * a:3 * a + 3]
        o = o_ref[...]
        te = te_ref[...]
        chosen = ch_ref[...][:, None, :]
        tf = (te[:, 0:1, :], te[:, 1:2, :], chosen)

        acc = None
        for k in range(K):
            d = o[:, k] - tf[k]
            sq = d * d
            acc = sq if acc is None else acc + sq

        s = jnp.dot(acc.reshape(BB * R, H), ones_h,
                    preferred_element_type=jnp.float32).reshape(BB, R, H)
        m = jnp.max(s, axis=1, keepdims=True)
        e = jnp.exp(s - m)
        w = e / jnp.sum(e, axis=1, keepdims=True)

        outs[a][...] = jnp.stack(
            [jnp.sum(o[:, k] * w, axis=1) for k in range(K)], axis=0)


def _tc_chunk(c, operands_o_te, chosen, prev_outs):
    nb = BC // BB
    boff = c * (BC // BB)

    o_spec = pl.BlockSpec((BB, K, R, H), lambda i: (i + boff, 0, 0, 0))
    t_spec = pl.BlockSpec((BB, KT, H), lambda i: (i + boff, 0, 0))
    ch_spec = pl.BlockSpec((BB, H), lambda i: (i, 0))
    any_spec = pl.BlockSpec(memory_space=pl.ANY)
    out_spec = pl.BlockSpec((K, BB, H), lambda i: (0, i + boff, 0))

    in_specs = []
    operands = []
    for a, (o, te) in enumerate(operands_o_te):
        in_specs += [o_spec, t_spec, ch_spec]
        operands += [o, te, chosen[a]]
    aliases = {}
    if prev_outs is not None:
        in_specs += [any_spec] * 4
        operands += list(prev_outs)
        aliases = {12: 0, 13: 1, 14: 2, 15: 3}

    out_shape = jax.ShapeDtypeStruct((K, B, H), jnp.float32)
    return pl.pallas_call(
        _fused_body,
        grid_spec=pl.GridSpec(
            grid=(nb,),
            in_specs=in_specs,
            out_specs=[out_spec] * 4,
        ),
        out_shape=[out_shape] * 4,
        input_output_aliases=aliases,
        compiler_params=pltpu.CompilerParams(
            dimension_semantics=("arbitrary",)),
    )(*operands)


def kernel(outputs_position, tests_position, candidates_position,
           outputs_type, tests_type, candidates_type,
           outputs_size, tests_size, candidates_size,
           outputs_color, tests_color, candidates_color,
           targets):
    tgt = targets.astype(jnp.int32)
    cands = tuple(c.reshape(B * C, H) for c in
                  (candidates_position, candidates_type,
                   candidates_size, candidates_color))
    operands_o_te = tuple(
        (jnp.transpose(o, (0, 2, 1, 3)), te)
        for o, te in ((outputs_position, tests_position),
                      (outputs_type, tests_type),
                      (outputs_size, tests_size),
                      (outputs_color, tests_color)))

    chosen = [_sc_gather(c * BC, tgt, cands) for c in range(NCH)]

    outs = None
    for c in range(NCH):
        outs = _tc_chunk(c, operands_o_te, chosen[c], outs)
    return tuple(jnp.transpose(x, (1, 0, 2)) for x in outs)


# speedup vs baseline: 1.0003x; 1.0003x over previous
import functools

import jax
import jax.numpy as jnp
from jax import lax
from jax.experimental import pallas as pl
from jax.experimental.pallas import tpu as pltpu
from jax.experimental.pallas import tpu_sc as plsc

B, R, KT, C, H = 4096, 8, 2, 8, 128
K = KT + 1
BB = 256
NC, NS, L = 2, 16, 16
NW = NC * NS
BPW = B // NW


def _sc_gather_body(tgt_hbm, c0, c1, c2, c3, o0, o1, o2, o3,
                    tgt_v, idx_v, r0, r1, r2, r3, sem):
    wid = lax.axis_index("s") * NC + lax.axis_index("c")
    base = wid * BPW
    pltpu.sync_copy(tgt_hbm.at[pl.ds(base, BPW)], tgt_v)
    for i in range(BPW // L):
        tv = tgt_v[pl.ds(L * i, L)]
        bb = lax.iota(jnp.int32, L) + (base + L * i)
        idx_v[pl.ds(L * i, L)] = bb * C + tv
    copies = [pltpu.async_copy(c.at[idx_v], r, sem)
              for c, r in ((c0, r0), (c1, r1), (c2, r2), (c3, r3))]
    for cp in copies:
        cp.wait()
    for r, o in ((r0, o0), (r1, o1), (r2, o2), (r3, o3)):
        pltpu.sync_copy(r, o.at[pl.ds(base, BPW)])


def _sc_gather(targets, cands):
    row = jax.ShapeDtypeStruct((B, H), jnp.float32)
    fn = pl.kernel(
        _sc_gather_body,
        mesh=plsc.VectorSubcoreMesh(core_axis_name="c", subcore_axis_name="s"),
        out_type=[row] * 4,
        scratch_types=[
            pltpu.VMEM((BPW,), jnp.int32),
            pltpu.VMEM((BPW,), jnp.int32),
            pltpu.VMEM((BPW, H), jnp.float32),
            pltpu.VMEM((BPW, H), jnp.float32),
            pltpu.VMEM((BPW, H), jnp.float32),
            pltpu.VMEM((BPW, H), jnp.float32),
            pltpu.SemaphoreType.DMA,
        ],
    )
    return fn(targets, *[c.reshape(B * C, H) for c in cands])


def _rtree(x, op):
    s1 = op(x[:, 0:4, :], x[:, 4:8, :])
    s2 = op(s1[:, 0:2, :], s1[:, 2:4, :])
    return op(s2[:, 0:1, :], s2[:, 1:2, :])


def _fused_body(*refs):
    outs = refs[12:]
    ones_h = jnp.full((H, H), -1.0 / (K * H), dtype=jnp.float32)

    for a in range(4):
        o_ref, te_ref, ch_ref = refs[3 * a:3 * a + 3]
        o = o_ref[...]
        te = te_ref[...]
        chosen = ch_ref[...][:, None, :]
        tf = (te[:, 0:1, :], te[:, 1:2, :], chosen)

        acc = None
        for k in range(K):
            d = o[:, k] - tf[k]
            sq = d * d
            acc = sq if acc is None else acc + sq

        s = jnp.dot(acc.reshape(BB * R, H), ones_h,
                    preferred_element_type=jnp.float32).reshape(BB, R, H)
        m = _rtree(s, jnp.maximum)
        e = jnp.exp(s - m)
        w = e / _rtree(e, jnp.add)

        outs[a][...] = jnp.stack(
            [_rtree(o[:, k] * w, jnp.add)[:, 0, :] for k in range(K)],
            axis=0)


def kernel(outputs_position, tests_position, candidates_position,
           outputs_type, tests_type, candidates_type,
           outputs_size, tests_size, candidates_size,
           outputs_color, tests_color, candidates_color,
           targets):
    nb = B // BB
    chosen = _sc_gather(targets.astype(jnp.int32),
                        (candidates_position, candidates_type,
                         candidates_size, candidates_color))

    o_spec = pl.BlockSpec((BB, K, R, H), lambda i: (i, 0, 0, 0))
    t_spec = pl.BlockSpec((BB, KT, H), lambda i: (i, 0, 0))
    ch_spec = pl.BlockSpec((BB, H), lambda i: (i, 0))
    out_spec = pl.BlockSpec((K, BB, H), lambda i: (0, i, 0))

    in_specs = []
    operands = []
    for a, (o, te) in enumerate(
            ((outputs_position, tests_position),
             (outputs_type, tests_type),
             (outputs_size, tests_size),
             (outputs_color, tests_color))):
        in_specs += [o_spec, t_spec, ch_spec]
        operands += [jnp.transpose(o, (0, 2, 1, 3)), te, chosen[a]]

    out_shape = jax.ShapeDtypeStruct((K, B, H), jnp.float32)
    grid_spec = pl.GridSpec(
        grid=(nb,),
        in_specs=in_specs,
        out_specs=[out_spec] * 4,
    )
    outs = pl.pallas_call(
        _fused_body,
        grid_spec=grid_spec,
        out_shape=[out_shape] * 4,
        compiler_params=pltpu.CompilerParams(
            dimension_semantics=("arbitrary",)),
    )(*operands)
    return tuple(jnp.transpose(x, (1, 0, 2)) for x in outs)


# speedup vs baseline: 1.0255x; 1.0252x over previous
import functools

import jax
import jax.numpy as jnp
from jax import lax
from jax.experimental import pallas as pl
from jax.experimental.pallas import tpu as pltpu
from jax.experimental.pallas import tpu_sc as plsc

B, R, KT, C, H = 4096, 8, 2, 8, 128
K = KT + 1
BB = 256
NC, NS, L = 2, 16, 16
NW = NC * NS
BPW = B // NW


def _sc_gather_body(tgt_hbm, c0, c1, c2, c3, o0, o1, o2, o3,
                    tgt_v, idx_v, r0, r1, r2, r3, sem):
    wid = lax.axis_index("s") * NC + lax.axis_index("c")
    base = wid * BPW
    pltpu.sync_copy(tgt_hbm.at[pl.ds(base, BPW)], tgt_v)
    for i in range(BPW // L):
        tv = tgt_v[pl.ds(L * i, L)]
        bb = lax.iota(jnp.int32, L) + (base + L * i)
        idx_v[pl.ds(L * i, L)] = bb * C + tv
    copies = [pltpu.async_copy(c.at[idx_v], r, sem)
              for c, r in ((c0, r0), (c1, r1), (c2, r2), (c3, r3))]
    for cp in copies:
        cp.wait()
    for r, o in ((r0, o0), (r1, o1), (r2, o2), (r3, o3)):
        pltpu.sync_copy(r, o.at[pl.ds(base, BPW)])


def _sc_gather(targets, cands):
    row = jax.ShapeDtypeStruct((B, H), jnp.float32)
    fn = pl.kernel(
        _sc_gather_body,
        mesh=plsc.VectorSubcoreMesh(core_axis_name="c", subcore_axis_name="s"),
        out_type=[row] * 4,
        scratch_types=[
            pltpu.VMEM((BPW,), jnp.int32),
            pltpu.VMEM((BPW,), jnp.int32),
            pltpu.VMEM((BPW, H), jnp.float32),
            pltpu.VMEM((BPW, H), jnp.float32),
            pltpu.VMEM((BPW, H), jnp.float32),
            pltpu.VMEM((BPW, H), jnp.float32),
            pltpu.SemaphoreType.DMA,
        ],
    )
    return fn(targets, *[c.reshape(B * C, H) for c in cands])


def _rtree(x, op):
    t = op(x, pltpu.roll(x, 4, axis=1))
    t = op(t, pltpu.roll(t, 2, axis=1))
    return op(t, pltpu.roll(t, 1, axis=1))


def _fused_body(*refs):
    outs = refs[12:]
    ones_h = jnp.full((H, H), -1.0 / (K * H), dtype=jnp.float32)

    for a in range(4):
        o_ref, te_ref, ch_ref = refs[3 * a:3 * a + 3]
        o = o_ref[...]
        te = te_ref[...]
        chosen = ch_ref[...][:, None, :]
        tf = (te[:, 0:1, :], te[:, 1:2, :], chosen)

        acc = None
        for k in range(K):
            d = o[:, k] - tf[k]
            sq = d * d
            acc = sq if acc is None else acc + sq

        s = jnp.dot(acc.reshape(BB * R, H), ones_h,
                    preferred_element_type=jnp.float32).reshape(BB, R, H)
        m = _rtree(s, jnp.maximum)
        e = jnp.exp(s - m)
        w = e / _rtree(e, jnp.add)

        outs[a][...] = jnp.stack(
            [_rtree(o[:, k] * w, jnp.add)[:, 0, :] for k in range(K)],
            axis=0)


def kernel(outputs_position, tests_position, candidates_position,
           outputs_type, tests_type, candidates_type,
           outputs_size, tests_size, candidates_size,
           outputs_color, tests_color, candidates_color,
           targets):
    nb = B // BB
    chosen = _sc_gather(targets.astype(jnp.int32),
                        (candidates_position, candidates_type,
                         candidates_size, candidates_color))

    o_spec = pl.BlockSpec((BB, K, R, H), lambda i: (i, 0, 0, 0))
    t_spec = pl.BlockSpec((BB, KT, H), lambda i: (i, 0, 0))
    ch_spec = pl.BlockSpec((BB, H), lambda i: (i, 0))
    out_spec = pl.BlockSpec((K, BB, H), lambda i: (0, i, 0))

    in_specs = []
    operands = []
    for a, (o, te) in enumerate(
            ((outputs_position, tests_position),
             (outputs_type, tests_type),
             (outputs_size, tests_size),
             (outputs_color, tests_color))):
        in_specs += [o_spec, t_spec, ch_spec]
        operands += [jnp.transpose(o, (0, 2, 1, 3)), te, chosen[a]]

    out_shape = jax.ShapeDtypeStruct((K, B, H), jnp.float32)
    grid_spec = pl.GridSpec(
        grid=(nb,),
        in_specs=in_specs,
        out_specs=[out_spec] * 4,
    )
    outs = pl.pallas_call(
        _fused_body,
        grid_spec=grid_spec,
        out_shape=[out_shape] * 4,
        compiler_params=pltpu.CompilerParams(
            dimension_semantics=("arbitrary",)),
    )(*operands)
    return tuple(jnp.transpose(x, (1, 0, 2)) for x in outs)


# speedup vs baseline: 1.0256x; 1.0002x over previous
import functools

import jax
import jax.numpy as jnp
from jax import lax
from jax.experimental import pallas as pl
from jax.experimental.pallas import tpu as pltpu
from jax.experimental.pallas import tpu_sc as plsc

B, R, KT, C, H = 4096, 8, 2, 8, 128
K = KT + 1
BB = 256
NC, NS, L = 2, 16, 16
NW = NC * NS
BPW = B // NW


def _sc_gather_body(tgt_hbm, c0, c1, c2, c3, o0,
                    tgt_v, idx_v, r0, r1, r2, r3, sem):
    wid = lax.axis_index("s") * NC + lax.axis_index("c")
    base = wid * BPW
    pltpu.sync_copy(tgt_hbm.at[pl.ds(base, BPW)], tgt_v)
    for i in range(BPW // L):
        tv = tgt_v[pl.ds(L * i, L)]
        bb = lax.iota(jnp.int32, L) + (base + L * i)
        idx_v[pl.ds(L * i, L)] = bb * C + tv
    copies = [pltpu.async_copy(c.at[idx_v], r, sem)
              for c, r in ((c0, r0), (c1, r1), (c2, r2), (c3, r3))]
    for cp in copies:
        cp.wait()
    for a, r in enumerate((r0, r1, r2, r3)):
        pltpu.sync_copy(r, o0.at[a, pl.ds(base, BPW)])


def _sc_gather(targets, cands):
    fn = pl.kernel(
        _sc_gather_body,
        mesh=plsc.VectorSubcoreMesh(core_axis_name="c", subcore_axis_name="s"),
        out_type=jax.ShapeDtypeStruct((4, B, H), jnp.float32),
        scratch_types=[
            pltpu.VMEM((BPW,), jnp.int32),
            pltpu.VMEM((BPW,), jnp.int32),
            pltpu.VMEM((BPW, H), jnp.float32),
            pltpu.VMEM((BPW, H), jnp.float32),
            pltpu.VMEM((BPW, H), jnp.float32),
            pltpu.VMEM((BPW, H), jnp.float32),
            pltpu.SemaphoreType.DMA,
        ],
    )
    return fn(targets, *[c.reshape(B * C, H) for c in cands])


def _rtree(x, op):
    t = op(x, pltpu.roll(x, 4, axis=1))
    t = op(t, pltpu.roll(t, 2, axis=1))
    return op(t, pltpu.roll(t, 1, axis=1))


def _fused_body(*refs):
    outs = refs[12:]
    ones_h = jnp.full((H, H), -1.0 / (K * H), dtype=jnp.float32)

    for a in range(4):
        o_ref, te_ref, ch_ref = refs[3 * a:3 * a + 3]
        o = o_ref[...]
        te = te_ref[...]
        chosen = ch_ref[0][:, None, :]
        tf = (te[:, 0:1, :], te[:, 1:2, :], chosen)

        acc = None
        for k in range(K):
            d = o[:, k] - tf[k]
            sq = d * d
            acc = sq if acc is None else acc + sq

        s = jnp.dot(acc.reshape(BB * R, H), ones_h,
                    preferred_element_type=jnp.float32).reshape(BB, R, H)
        m = _rtree(s, jnp.maximum)
        e = jnp.exp(s - m)
        w = e / _rtree(e, jnp.add)

        outs[a][...] = jnp.stack(
            [_rtree(o[:, k] * w, jnp.add)[:, 0, :] for k in range(K)],
            axis=0)


def kernel(outputs_position, tests_position, candidates_position,
           outputs_type, tests_type, candidates_type,
           outputs_size, tests_size, candidates_size,
           outputs_color, tests_color, candidates_color,
           targets):
    nb = B // BB
    chosen = _sc_gather(targets.astype(jnp.int32),
                        (candidates_position, candidates_type,
                         candidates_size, candidates_color))

    o_spec = pl.BlockSpec((BB, K, R, H), lambda i: (i, 0, 0, 0))
    t_spec = pl.BlockSpec((BB, KT, H), lambda i: (i, 0, 0))
    def ch_spec(a):
        return pl.BlockSpec((1, BB, H), lambda i, a=a: (a, i, 0))
    out_spec = pl.BlockSpec((K, BB, H), lambda i: (0, i, 0))

    in_specs = []
    operands = []
    for a, (o, te) in enumerate(
            ((outputs_position, tests_position),
             (outputs_type, tests_type),
             (outputs_size, tests_size),
             (outputs_color, tests_color))):
        in_specs += [o_spec, t_spec, ch_spec(a)]
        operands += [jnp.transpose(o, (0, 2, 1, 3)), te, chosen]

    out_shape = jax.ShapeDtypeStruct((K, B, H), jnp.float32)
    grid_spec = pl.GridSpec(
        grid=(nb,),
        in_specs=in_specs,
        out_specs=[out_spec] * 4,
    )
    outs = pl.pallas_call(
        _fused_body,
        grid_spec=grid_spec,
        out_shape=[out_shape] * 4,
        compiler_params=pltpu.CompilerParams(
            dimension_semantics=("arbitrary",)),
    )(*operands)
    return tuple(jnp.transpose(x, (1, 0, 2)) for x in outs)


# speedup vs baseline: 1.0267x; 1.0010x over previous
import jax
import jax.numpy as jnp
from jax import lax
from jax.experimental import pallas as pl
from jax.experimental.pallas import tpu as pltpu
from jax.experimental.pallas import tpu_sc as plsc

B, R, KT, C, H = 4096, 8, 2, 8, 128
K = KT + 1
BB = 256
NC, NS, L = 2, 16, 16
NW = NC * NS
BPW = B // NW


def _sc_gather_body(tgt_hbm, c0, c1, c2, c3, o0,
                    tgt_v, idx_v, r0, r1, r2, r3, sem):
    wid = lax.axis_index("s") * NC + lax.axis_index("c")
    base = wid * BPW
    pltpu.sync_copy(tgt_hbm.at[pl.ds(base, BPW)], tgt_v)
    for i in range(BPW // L):
        tv = tgt_v[pl.ds(L * i, L)]
        bb = lax.iota(jnp.int32, L) + (base + L * i)
        idx_v[pl.ds(L * i, L)] = bb * C + tv
    copies = [pltpu.async_copy(c.at[idx_v], r, sem)
              for c, r in ((c0, r0), (c1, r1), (c2, r2), (c3, r3))]
    for cp in copies:
        cp.wait()
    for a, r in enumerate((r0, r1, r2, r3)):
        pltpu.sync_copy(r, o0.at[a, pl.ds(base, BPW)])


def _sc_gather(targets, cands):
    fn = pl.kernel(
        _sc_gather_body,
        mesh=plsc.VectorSubcoreMesh(core_axis_name="c", subcore_axis_name="s"),
        out_type=jax.ShapeDtypeStruct((4, B, H), jnp.float32),
        scratch_types=[
            pltpu.VMEM((BPW,), jnp.int32),
            pltpu.VMEM((BPW,), jnp.int32),
            pltpu.VMEM((BPW, H), jnp.float32),
            pltpu.VMEM((BPW, H), jnp.float32),
            pltpu.VMEM((BPW, H), jnp.float32),
            pltpu.VMEM((BPW, H), jnp.float32),
            pltpu.SemaphoreType.DMA,
        ],
    )
    return fn(targets, *[c.reshape(B * C, H) for c in cands])


def _rtree(x, op):
    t = op(x, pltpu.roll(x, 4, axis=1))
    t = op(t, pltpu.roll(t, 2, axis=1))
    return op(t, pltpu.roll(t, 1, axis=1))


def _fused_body(*refs):
    outs = refs[12:]
    ones_h = jnp.full((H, H), -1.0 / (K * H), dtype=jnp.float32)

    for a in range(4):
        o_ref, te_ref, ch_ref = refs[3 * a:3 * a + 3]
        o = o_ref[...]
        te = te_ref[...]
        chosen = ch_ref[0][:, None, :]
        tf = (te[:, 0:1, :], te[:, 1:2, :], chosen)

        acc = None
        for k in range(K):
            d = o[:, k] - tf[k]
            sq = d * d
            acc = sq if acc is None else acc + sq

        s = jnp.dot(acc.reshape(BB * R, H), ones_h,
                    preferred_element_type=jnp.float32).reshape(BB, R, H)
        m = _rtree(s, jnp.maximum)
        e = jnp.exp(s - m)
        w = e / _rtree(e, jnp.add)

        outs[a][...] = jnp.stack(
            [_rtree(o[:, k] * w, jnp.add)[:, 0, :] for k in range(K)],
            axis=0)


def kernel(outputs_position, tests_position, candidates_position,
           outputs_type, tests_type, candidates_type,
           outputs_size, tests_size, candidates_size,
           outputs_color, tests_color, candidates_color,
           targets):
    nb = B // BB
    chosen = _sc_gather(targets.astype(jnp.int32),
                        (candidates_position, candidates_type,
                         candidates_size, candidates_color))

    o_spec = pl.BlockSpec((BB, K, R, H), lambda i: (i, 0, 0, 0))
    t_spec = pl.BlockSpec((BB, KT, H), lambda i: (i, 0, 0))
    def ch_spec(a):
        return pl.BlockSpec((1, BB, H), lambda i, a=a: (a, i, 0))
    out_spec = pl.BlockSpec((K, BB, H), lambda i: (0, i, 0))

    in_specs = []
    operands = []
    for a, (o, te) in enumerate(
            ((outputs_position, tests_position),
             (outputs_type, tests_type),
             (outputs_size, tests_size),
             (outputs_color, tests_color))):
        in_specs += [o_spec, t_spec, ch_spec(a)]
        operands += [jnp.transpose(o, (0, 2, 1, 3)), te, chosen]

    out_shape = jax.ShapeDtypeStruct((K, B, H), jnp.float32)
    grid_spec = pl.GridSpec(
        grid=(nb,),
        in_specs=in_specs,
        out_specs=[out_spec] * 4,
    )
    outs = pl.pallas_call(
        _fused_body,
        grid_spec=grid_spec,
        out_shape=[out_shape] * 4,
        compiler_params=pltpu.CompilerParams(
            dimension_semantics=("arbitrary",)),
    )(*operands)
    return tuple(jnp.transpose(x, (1, 0, 2)) for x in outs)
